# Initial kernel scaffold; baseline (speedup 1.0000x reference)
import jax
import jax.numpy as jnp
from jax.experimental import pallas as pl


def kernel(input_ids, token_type_ids, word_table, pos_table, tt_table, gamma, beta):
    raise NotImplementedError("write your pallas kernel here")


# speedup vs baseline: 1.9160x; 1.9160x over previous
import functools

import jax
import jax.numpy as jnp
from jax import lax
from jax.experimental import pallas as pl
from jax.experimental.pallas import tpu as pltpu
from jax.experimental.pallas import tpu_sc as plsc

_B, _T, _H = 64, 512, 768
_N = _B * _T
_EPS = 1e-12

_NC, _NS = 2, 16
_NW = _NC * _NS
_RPW = _N // _NW
_CHUNK = 128
_NCHUNK = _RPW // _CHUNK


def _sc_gather(word_table, ids):
    mesh = plsc.VectorSubcoreMesh(
        core_axis_name="c", subcore_axis_name="s",
        num_cores=_NC, num_subcores=_NS)

    @functools.partial(
        pl.kernel,
        out_type=jax.ShapeDtypeStruct((_N, _H), jnp.float32),
        mesh=mesh,
        scratch_types=[
            pltpu.VMEM((_CHUNK,), jnp.int32),
            pltpu.VMEM((_CHUNK, _H), jnp.float32),
            pltpu.SemaphoreType.DMA,
        ],
    )
    def k(word_hbm, ids_hbm, out_hbm, idx_v, rows_v, sem):
        wid = lax.axis_index("s") * _NC + lax.axis_index("c")
        base = wid * _RPW
        for c in range(_NCHUNK):
            off = base + c * _CHUNK
            pltpu.sync_copy(ids_hbm.at[pl.ds(off, _CHUNK)], idx_v)
            pltpu.async_copy(word_hbm.at[idx_v], rows_v, sem).wait()
            pltpu.sync_copy(rows_v, out_hbm.at[pl.ds(off, _CHUNK)])

    return k(word_table, ids)


def _tc_body(wemb_ref, tt_ids_ref, pos_ref, tt_ref, g_ref, b_ref, out_ref):
    x = wemb_ref[...]
    m = tt_ids_ref[...] == 1
    tt0 = tt_ref[0:1, :]
    tt1 = tt_ref[1:2, :]
    x = x + pos_ref[...] + jnp.where(m, tt1, tt0)
    mean = jnp.mean(x, axis=-1, keepdims=True)
    xc = x - mean
    var = jnp.mean(xc * xc, axis=-1, keepdims=True)
    y = xc * lax.rsqrt(var + _EPS)
    out_ref[...] = y * g_ref[...] + b_ref[...]


def _tc_add_ln(wemb, tt_ids, pos_table, tt_table, gamma, beta):
    return pl.pallas_call(
        _tc_body,
        grid=(_B,),
        in_specs=[
            pl.BlockSpec((_T, _H), lambda i: (i, 0)),
            pl.BlockSpec((_T, 1), lambda i: (i, 0)),
            pl.BlockSpec((_T, _H), lambda i: (0, 0)),
            pl.BlockSpec((2, _H), lambda i: (0, 0)),
            pl.BlockSpec((1, _H), lambda i: (0, 0)),
            pl.BlockSpec((1, _H), lambda i: (0, 0)),
        ],
        out_specs=pl.BlockSpec((_T, _H), lambda i: (i, 0)),
        out_shape=jax.ShapeDtypeStruct((_N, _H), jnp.float32),
    )(wemb, tt_ids, pos_table, tt_table, gamma, beta)


def kernel(input_ids, token_type_ids, word_table, pos_table, tt_table, gamma, beta):
    ids = input_ids.reshape(-1).astype(jnp.int32)
    tt_ids = token_type_ids.reshape(-1, 1).astype(jnp.int32)
    wemb = _sc_gather(word_table, ids)
    y = _tc_add_ln(wemb, tt_ids, pos_table, tt_table,
                   gamma.reshape(1, _H), beta.reshape(1, _H))
    return y.reshape(_B, _T, _H)


# speedup vs baseline: 1.9690x; 1.0277x over previous
import functools

import jax
import jax.numpy as jnp
from jax import lax
from jax.experimental import pallas as pl
from jax.experimental.pallas import tpu as pltpu
from jax.experimental.pallas import tpu_sc as plsc

_B, _T, _H = 64, 512, 768
_N = _B * _T
_EPS = 1e-12

_S = 4
_BS = _B // _S
_NS_TOK = _N // _S

_NC, _NSC = 2, 16
_NW = _NC * _NSC
_RPW = _NS_TOK // _NW
_CHUNK = 128
_NCHUNK = _RPW // _CHUNK


def _sc_gather(word_table, ids):
    mesh = plsc.VectorSubcoreMesh(
        core_axis_name="c", subcore_axis_name="s",
        num_cores=_NC, num_subcores=_NSC)

    @functools.partial(
        pl.kernel,
        out_type=jax.ShapeDtypeStruct((_NS_TOK, _H), jnp.float32),
        mesh=mesh,
        scratch_types=[
            pltpu.VMEM((_CHUNK,), jnp.int32),
            pltpu.VMEM((_CHUNK, _H), jnp.float32),
            pltpu.SemaphoreType.DMA,
        ],
    )
    def k(word_hbm, ids_hbm, out_hbm, idx_v, rows_v, sem):
        wid = lax.axis_index("s") * _NC + lax.axis_index("c")
        base = wid * _RPW
        for c in range(_NCHUNK):
            off = base + c * _CHUNK
            pltpu.sync_copy(ids_hbm.at[pl.ds(off, _CHUNK)], idx_v)
            pltpu.async_copy(word_hbm.at[idx_v], rows_v, sem).wait()
            pltpu.sync_copy(rows_v, out_hbm.at[pl.ds(off, _CHUNK)])

    return k(word_table, ids)


def _tc_body(wemb_ref, tt_ids_ref, pos_ref, tt_ref, g_ref, b_ref, out_ref):
    x = wemb_ref[...]
    m = tt_ids_ref[...] == 1
    x = x + pos_ref[...] + jnp.where(m, tt_ref[1:2, :], tt_ref[0:1, :])
    mean = jnp.mean(x, axis=-1, keepdims=True)
    xc = x - mean
    var = jnp.mean(xc * xc, axis=-1, keepdims=True)
    y = xc * lax.rsqrt(var + _EPS)
    out_ref[...] = y * g_ref[...] + b_ref[...]


def _tc_body_acc(y_ref, wemb_ref, tt_ids_ref, pos_ref, tt_ref, g_ref, b_ref,
                 out_ref):
    del y_ref
    _tc_body(wemb_ref, tt_ids_ref, pos_ref, tt_ref, g_ref, b_ref, out_ref)


_DENSE_SPECS = [
    pl.BlockSpec((_T, _H), lambda i: (i, 0)),
    pl.BlockSpec((_T, 1), lambda i: (i, 0)),
    pl.BlockSpec((_T, _H), lambda i: (0, 0)),
    pl.BlockSpec((2, _H), lambda i: (0, 0)),
    pl.BlockSpec((1, _H), lambda i: (0, 0)),
    pl.BlockSpec((1, _H), lambda i: (0, 0)),
]


def _tc_add_ln_slice(y, wemb_s, tt_ids_s, pos, tt, g, b, s):
    out_spec = pl.BlockSpec((_T, _H), lambda i, s=s: (s * _BS + i, 0))
    if y is None:
        return pl.pallas_call(
            _tc_body,
            grid=(_BS,),
            in_specs=_DENSE_SPECS,
            out_specs=out_spec,
            out_shape=jax.ShapeDtypeStruct((_N, _H), jnp.float32),
        )(wemb_s, tt_ids_s, pos, tt, g, b)
    return pl.pallas_call(
        _tc_body_acc,
        grid=(_BS,),
        in_specs=[pl.BlockSpec(memory_space=pl.ANY)] + _DENSE_SPECS,
        out_specs=out_spec,
        out_shape=jax.ShapeDtypeStruct((_N, _H), jnp.float32),
        input_output_aliases={0: 0},
    )(y, wemb_s, tt_ids_s, pos, tt, g, b)


def kernel(input_ids, token_type_ids, word_table, pos_table, tt_table, gamma, beta):
    ids = input_ids.reshape(-1).astype(jnp.int32)
    tt_ids = token_type_ids.reshape(-1, 1).astype(jnp.int32)
    g = gamma.reshape(1, _H)
    b = beta.reshape(1, _H)

    wembs = [_sc_gather(word_table, ids[s * _NS_TOK:(s + 1) * _NS_TOK])
             for s in range(_S)]
    y = None
    for s in range(_S):
        tt_s = tt_ids[s * _NS_TOK:(s + 1) * _NS_TOK]
        y = _tc_add_ln_slice(y, wembs[s], tt_s, pos_table, tt_table, g, b, s)
    return y.reshape(_B, _T, _H)
